# baseline (speedup 1.0000x reference)
import functools

import jax
import jax.numpy as jnp
from jax import lax
from jax.experimental import pallas as pl
from jax.experimental.pallas import tpu as pltpu
from jax.experimental.pallas import tpu_sc as plsc

N = 10000
E = 320000
D = 128
EP = 327680
EPH = EP // 2
NPAD = 10240
NC = 2
NS = 16
GCH = 128

BLK_E = 2048
BLK_N = 2000

_F32 = jnp.float32


def _dot(a, b):
    return jnp.dot(a, b, preferred_element_type=_F32)


def _ln(x, g, b):
    m = jnp.mean(x, axis=-1, keepdims=True)
    xc = x - m
    v = jnp.mean(xc * xc, axis=-1, keepdims=True)
    return xc / jnp.sqrt(v + 1e-5) * g + b


def _full(shape):
    nd = len(shape)
    return pl.BlockSpec(shape, lambda i: (0,) * nd)


def _mlp3_ln_body(x, w1, b1, w2, b2, w3, b3, g, b, out):
    h = jnp.maximum(_dot(x[...], w1[...]) + b1[...], 0.0)
    h = jnp.maximum(_dot(h, w2[...]) + b2[...], 0.0)
    h = _dot(h, w3[...]) + b3[...]
    out[...] = _ln(h, g[...], b[...])


def _mlp3_ln(x, mlp, lng, lnb, blk):
    (w1, b1), (w2, b2), (w3, b3) = mlp
    n, din = x.shape
    return pl.pallas_call(
        _mlp3_ln_body,
        grid=(n // blk,),
        in_specs=[
            pl.BlockSpec((blk, din), lambda i: (i, 0)),
            _full((din, D)), _full((1, D)),
            _full((D, D)), _full((1, D)),
            _full((D, D)), _full((1, D)),
            _full((1, D)), _full((1, D)),
        ],
        out_specs=pl.BlockSpec((blk, D), lambda i: (i, 0)),
        out_shape=jax.ShapeDtypeStruct((n, D), _F32),
    )(x, w1, b1.reshape(1, D), w2, b2.reshape(1, D), w3, b3.reshape(1, D),
      lng.reshape(1, D), lnb.reshape(1, D))


def _edge_step_body(gs, gr, el, w1, b1, w2, b2, w3, b3, g, b, ne_out, el_out):
    x = jnp.concatenate([gs[...], gr[...], el[...]], axis=-1)
    x = _dot(x, w1[...]) + b1[...]
    x = jnp.maximum(x, 0.0)
    x = jnp.maximum(_dot(x, w2[...]) + b2[...], 0.0)
    x = _dot(x, w3[...]) + b3[...]
    ne = _ln(x, g[...], b[...])
    ne_out[...] = ne
    el_out[...] = el[...] + ne


def _edge_step(gs, gr, el, w1, b1, w2, b2, w3, b3, lng, lnb):
    n = gs.shape[0]
    row = pl.BlockSpec((BLK_E, D), lambda i: (i, 0))
    return pl.pallas_call(
        _edge_step_body,
        grid=(n // BLK_E,),
        in_specs=[row, row, row,
                  _full((3 * D, D)), _full((1, D)),
                  _full((D, D)), _full((1, D)),
                  _full((D, D)), _full((1, D)),
                  _full((1, D)), _full((1, D))],
        out_specs=[row, row],
        out_shape=[jax.ShapeDtypeStruct((n, D), _F32)] * 2,
    )(gs, gr, el, w1, b1.reshape(1, D), w2, b2.reshape(1, D),
      w3, b3.reshape(1, D), lng.reshape(1, D), lnb.reshape(1, D))


def _node_step_body(nl, a0, a1, a2, a3, wna, wnb, b1, w2, b2, w3, b3, g, b,
                    out):
    nlv = nl[...]
    agg = (a0[...] + a1[...]) + (a2[...] + a3[...])
    x = _dot(nlv, wna[...]) + _dot(agg, wnb[...]) + b1[...]
    x = jnp.maximum(x, 0.0)
    x = jnp.maximum(_dot(x, w2[...]) + b2[...], 0.0)
    x = _dot(x, w3[...]) + b3[...]
    out[...] = nlv + _ln(x, g[...], b[...])


def _node_step(node_lat, a0, a1, a2, a3, wna, wnb, b1, w2, b2, w3, b3,
               lng, lnb):
    row = pl.BlockSpec((BLK_N, D), lambda i: (i, 0))
    return pl.pallas_call(
        _node_step_body,
        grid=(N // BLK_N,),
        in_specs=[row, row, row, row, row,
                  _full((D, D)), _full((D, D)), _full((1, D)),
                  _full((D, D)), _full((1, D)),
                  _full((D, D)), _full((1, D)),
                  _full((1, D)), _full((1, D))],
        out_specs=row,
        out_shape=jax.ShapeDtypeStruct((N, D), _F32),
    )(node_lat, a0, a1, a2, a3, wna, wnb, b1.reshape(1, D),
      w2, b2.reshape(1, D), w3, b3.reshape(1, D), lng.reshape(1, D),
      lnb.reshape(1, D))


def _dec_body(x, w1, b1, w2, b2, w3, b3, out):
    h = jnp.maximum(_dot(x[...], w1[...]) + b1[...], 0.0)
    h = jnp.maximum(_dot(h, w2[...]) + b2[...], 0.0)
    out[...] = _dot(h, w3[...]) + b3[...]


def _decoder(node_lat, mlp):
    (w1, b1), (w2, b2), (w3, b3) = mlp
    dout = w3.shape[1]
    w3p = jnp.zeros((D, D), _F32).at[:, :dout].set(w3)
    b3p = jnp.zeros((1, D), _F32).at[0, :dout].set(b3)
    row = pl.BlockSpec((BLK_N, D), lambda i: (i, 0))
    full = pl.pallas_call(
        _dec_body,
        grid=(N // BLK_N,),
        in_specs=[row,
                  _full((D, D)), _full((1, D)),
                  _full((D, D)), _full((1, D)),
                  _full((D, D)), _full((1, D))],
        out_specs=row,
        out_shape=jax.ShapeDtypeStruct((N, D), _F32),
    )(node_lat, w1, b1.reshape(1, D), w2, b2.reshape(1, D), w3p, b3p)
    return full[:, :dout]


TCHUNKS = (EP // GCH) // NS
SITERS = (EPH // (NC * NS)) // GCH


def _sc_gather_body(ps_hbm, pr_hbm, snd_hbm, rcv_hbm,
                    gsa_hbm, gra_hbm, gsb_hbm, grb_hbm,
                    tab_sh, idx, rows,
                    sem_i0, sem_i1, sem_g, sem_w0, sem_w1):
    cid = lax.axis_index("c")
    sid = lax.axis_index("s")

    def stage(tab_hbm):
        @pl.when(sid < NS - 1)
        def _():
            pltpu.sync_copy(tab_hbm.at[pl.ds(sid * 640, 640)],
                            tab_sh.at[pl.ds(sid * 640, 640)])

        @pl.when(sid == NS - 1)
        def _():
            pltpu.sync_copy(tab_hbm.at[pl.ds(9600, 400)],
                            tab_sh.at[pl.ds(9600, 400)])

    @pl.when(cid == 0)
    def _():
        stage(ps_hbm)

    @pl.when(cid == 1)
    def _():
        stage(pr_hbm)

    plsc.subcore_barrier()

    t0 = sid * TCHUNKS
    sem_i = (sem_i0, sem_i1)
    sem_w = (sem_w0, sem_w1)

    def pipe(idx_hbm, out_hbm, base):
        pltpu.async_copy(idx_hbm.at[t0], idx.at[0], sem_i0)

        def outer(g, carry):
            for b in range(2):
                nb = 1 - b
                c = 2 * g + b
                off = base + c * GCH
                pltpu.make_async_copy(idx_hbm.at[t0 + c], idx.at[b],
                                      sem_i[b]).wait()
                @pl.when(c >= 2)
                def _():
                    poff = off - 2 * GCH
                    pltpu.make_async_copy(rows.at[b],
                                          out_hbm.at[pl.ds(poff, GCH)],
                                          sem_w[b]).wait()
                a = pltpu.async_copy(tab_sh.at[idx.at[b]], rows.at[b],
                                     sem_g)
                @pl.when(c + 1 < TCHUNKS)
                def _():
                    pltpu.async_copy(idx_hbm.at[t0 + c + 1], idx.at[nb],
                                     sem_i[nb])
                a.wait()
                pltpu.async_copy(rows.at[b], out_hbm.at[pl.ds(off, GCH)],
                                 sem_w[b])
            return carry

        lax.fori_loop(0, TCHUNKS // 2, outer, 0)
        for b in range(2):
            off = base + (TCHUNKS - 2 + b) * GCH
            pltpu.make_async_copy(rows.at[b], out_hbm.at[pl.ds(off, GCH)],
                                  sem_w[b]).wait()

    half_b = sid >= NS // 2
    base_a = t0 * GCH
    base_b = t0 * GCH - EPH

    @pl.when(cid == 0)
    def _():
        @pl.when(jnp.logical_not(half_b))
        def _():
            pipe(snd_hbm, gsa_hbm, base_a)

        @pl.when(half_b)
        def _():
            pipe(snd_hbm, gsb_hbm, base_b)

    @pl.when(cid == 1)
    def _():
        @pl.when(jnp.logical_not(half_b))
        def _():
            pipe(rcv_hbm, gra_hbm, base_a)

        @pl.when(half_b)
        def _():
            pipe(rcv_hbm, grb_hbm, base_b)


def _sc_scatter_body(ne_hbm, rcv_hbm, zeros_hbm, out_hbm, idx_v, rows_v,
                     agg_sh, sem_l0, sem_l1, sem_s0, sem_s1):
    cid = lax.axis_index("c")
    sid = lax.axis_index("s")
    rpt = NPAD // NS
    pltpu.sync_copy(zeros_hbm.at[pl.ds(sid * rpt, rpt)],
                    agg_sh.at[pl.ds(sid * rpt, rpt)])
    plsc.subcore_barrier()
    t0 = cid * ((EPH // NC) // GCH) + sid * SITERS
    base = t0 * GCH
    sem_l = (sem_l0, sem_l1)
    sem_s = (sem_s0, sem_s1)

    pltpu.async_copy(rcv_hbm.at[t0], idx_v.at[0], sem_l0)
    pltpu.async_copy(ne_hbm.at[pl.ds(base, GCH)], rows_v.at[0], sem_l0)

    def outer(g, carry):
        for b in range(2):
            nb = 1 - b
            c = 2 * g + b
            off = base + c * GCH
            pltpu.make_async_copy(rcv_hbm.at[t0 + c], idx_v.at[b],
                                  sem_l[b]).wait()
            pltpu.make_async_copy(ne_hbm.at[pl.ds(off, GCH)], rows_v.at[b],
                                  sem_l[b]).wait()
            pltpu.async_copy(rows_v.at[b], agg_sh.at[idx_v.at[b]], sem_s[b],
                             add=True)
            @pl.when(c >= 1)
            def _():
                pltpu.make_async_copy(rows_v.at[nb],
                                      agg_sh.at[idx_v.at[nb]],
                                      sem_s[nb]).wait()
            @pl.when(c + 1 < SITERS)
            def _():
                noff = off + GCH
                pltpu.async_copy(rcv_hbm.at[t0 + c + 1], idx_v.at[nb],
                                 sem_l[nb])
                pltpu.async_copy(ne_hbm.at[pl.ds(noff, GCH)], rows_v.at[nb],
                                 sem_l[nb])
        return carry

    lax.fori_loop(0, SITERS // 2, outer, 0)
    pltpu.make_async_copy(rows_v.at[1], agg_sh.at[idx_v.at[1]],
                          sem_s[1]).wait()
    plsc.subcore_barrier()
    pltpu.sync_copy(agg_sh.at[pl.ds(sid * rpt, rpt)],
                    out_hbm.at[pl.ds(cid * NPAD + sid * rpt, rpt)])


@functools.cache
def _sc_calls():
    mesh = plsc.VectorSubcoreMesh(core_axis_name="c", subcore_axis_name="s")
    gather = pl.kernel(
        _sc_gather_body,
        out_type=(jax.ShapeDtypeStruct((EPH, D), _F32),
                  jax.ShapeDtypeStruct((EPH, D), _F32),
                  jax.ShapeDtypeStruct((EPH, D), _F32),
                  jax.ShapeDtypeStruct((EPH, D), _F32)),
        mesh=mesh,
        scratch_types=[
            pltpu.VMEM_SHARED((N, D), _F32),
            pltpu.VMEM((2, GCH), jnp.int32),
            pltpu.VMEM((2, GCH, D), _F32),
            pltpu.SemaphoreType.DMA,
            pltpu.SemaphoreType.DMA,
            pltpu.SemaphoreType.DMA,
            pltpu.SemaphoreType.DMA,
            pltpu.SemaphoreType.DMA,
        ],
    )
    scatter = pl.kernel(
        _sc_scatter_body,
        out_type=jax.ShapeDtypeStruct((2 * NPAD, D), _F32),
        mesh=mesh,
        scratch_types=[
            pltpu.VMEM((2, GCH), jnp.int32),
            pltpu.VMEM((2, GCH, D), _F32),
            pltpu.VMEM_SHARED((NPAD, D), _F32),
            pltpu.SemaphoreType.DMA,
            pltpu.SemaphoreType.DMA,
            pltpu.SemaphoreType.DMA,
            pltpu.SemaphoreType.DMA,
        ],
    )
    return gather, scatter


def _gather_on_sc(ps, pr, snd, rcv):
    return _sc_calls()[0](ps, pr, snd, rcv)


def _scatter_on_sc(ne, rcv, zeros):
    return _sc_calls()[1](ne, rcv, zeros)


def kernel(node_features, edge_features, senders, receivers, params):
    p = params

    pad = EP - E
    senders_p = jnp.concatenate(
        [senders, jnp.zeros((pad,), jnp.int32)]).reshape(EP // GCH, GCH)
    recv_gather_p = jnp.concatenate(
        [receivers, jnp.zeros((pad,), jnp.int32)]).reshape(EP // GCH, GCH)
    recv_scatter_p = jnp.concatenate(
        [receivers, jnp.full((pad,), N, jnp.int32)]).reshape(EP // GCH, GCH)
    ef_p = jnp.concatenate(
        [edge_features, jnp.zeros((pad, edge_features.shape[1]), _F32)])
    zeros_acc = jnp.zeros((NPAD, D), _F32)

    node_lat = _mlp3_ln(node_features, p['enc_node'], *p['enc_node_ln'],
                        blk=BLK_N)
    edge_lat = _mlp3_ln(ef_p, p['enc_edge'], *p['enc_edge_ln'], blk=BLK_E)

    hch = EPH // GCH
    rcvs_h = (recv_scatter_p[:hch], recv_scatter_p[hch:])
    el_h = [edge_lat[:EPH], edge_lat[EPH:]]

    for sp in p['steps']:
        (w1, b1), (w2, b2), (w3, b3) = sp['edge_mlp']
        aggs = []
        gsa, gra, gsb, grb = _gather_on_sc(node_lat, node_lat, senders_p,
                                           recv_gather_p)
        g_h = [(gsa, gra), (gsb, grb)]
        for h in range(2):
            gs, gr = g_h[h]
            new_e, el_h[h] = _edge_step(gs, gr, el_h[h], w1, b1, w2, b2,
                                        w3, b3, *sp['edge_ln'])
            aggs.append(_scatter_on_sc(new_e, rcvs_h[h], zeros_acc))
        (wn1, bn1), (wn2, bn2), (wn3, bn3) = sp['node_mlp']
        node_lat = _node_step(node_lat,
                              aggs[0][:N], aggs[0][NPAD:NPAD + N],
                              aggs[1][:N], aggs[1][NPAD:NPAD + N],
                              wn1[:D], wn1[D:], bn1, wn2, bn2, wn3, bn3,
                              *sp['node_ln'])

    return _decoder(node_lat, p['dec'])
